# Initial kernel scaffold; baseline (speedup 1.0000x reference)
import jax
import jax.numpy as jnp
from jax.experimental import pallas as pl


def kernel(x, edge_index, W1, b1, W2, b2, W3, b3):
    raise NotImplementedError("write your pallas kernel here")


# speedup vs baseline: 26.3528x; 26.3528x over previous
import functools

import jax
import jax.numpy as jnp
from jax import lax
from jax.experimental import pallas as pl
from jax.experimental.pallas import tpu as pltpu
from jax.experimental.pallas import tpu_sc as plsc

_N = 10000
_E = 320000
_D = 128
_W = 125
_ROWS = _E // _W
_RPC = _ROWS // 2
_LPT = _RPC // 16
_NPT = _N // 16


def _sc_mesh():
    return plsc.VectorSubcoreMesh(core_axis_name="c", subcore_axis_name="s")


def _deg_sc(dst2d):
    @functools.partial(
        pl.kernel,
        out_type=jax.ShapeDtypeStruct((2, _N, 16), jnp.float32),
        mesh=_sc_mesh(),
        scratch_types=[
            pltpu.VMEM_SHARED((_N, 16), jnp.float32),
            pltpu.VMEM((_LPT, _W), jnp.int32),
            pltpu.VMEM((_W, 16), jnp.float32),
            pltpu.VMEM((125, 16), jnp.float32),
        ],
    )
    def k(dst_hbm, out_hbm, dacc, didx, ones, zbuf):
        c = lax.axis_index("c")
        s = lax.axis_index("s")
        zv = jnp.zeros((16,), jnp.float32)

        @pl.loop(0, 125)
        def _(i):
            zbuf[i, pl.ds(0, 16)] = zv

        @pl.loop(0, 5)
        def _(t):
            pltpu.sync_copy(zbuf, dacc.at[pl.ds(s * _NPT + t * 125, 125)])

        ov = jnp.full((16,), 1.0, jnp.float32)

        @pl.loop(0, _W)
        def _(i):
            ones[i, pl.ds(0, 16)] = ov

        rb = c * _RPC + s * _LPT
        pltpu.sync_copy(dst_hbm.at[pl.ds(rb, _LPT)], didx)
        plsc.subcore_barrier()

        @pl.loop(0, _LPT)
        def _(j):
            pltpu.sync_copy(ones, dacc.at[didx.at[j]], add=True)

        plsc.subcore_barrier()

        @pl.when(s < 15)
        def _():
            pltpu.sync_copy(dacc.at[pl.ds(s * 624, 624)],
                            out_hbm.at[c, pl.ds(s * 624, 624)])

        @pl.when(s == 15)
        def _():
            pltpu.sync_copy(dacc.at[pl.ds(9360, 640)],
                            out_hbm.at[c, pl.ds(9360, 640)])

    return k(dst2d)


def _agg_sc(u, src2d, dst2d):
    ph = 2
    cpp = _LPT // ph

    @functools.partial(
        pl.kernel,
        out_type=jax.ShapeDtypeStruct((2, _N, _D), jnp.float32),
        mesh=_sc_mesh(),
        scratch_types=[
            pltpu.VMEM_SHARED((_N, _D), jnp.float32),
            pltpu.VMEM((cpp, _W), jnp.int32),
            pltpu.VMEM((cpp, _W), jnp.int32),
            pltpu.VMEM((_W, _D), jnp.float32),
            pltpu.VMEM((_W, _D), jnp.float32),
            pltpu.SemaphoreType.DMA,
            pltpu.SemaphoreType.DMA,
        ],
    )
    def k(u_hbm, s_hbm, d_hbm, out_hbm, acc, sidx, didx, r0, r1, sem0, sem1):
        c = lax.axis_index("c")
        s = lax.axis_index("s")
        zv = jnp.zeros((16,), jnp.float32)

        @pl.loop(0, _W)
        def _(i):
            for jj in range(8):
                r0[i, pl.ds(jj * 16, 16)] = zv

        @pl.loop(0, 5)
        def _(t):
            pltpu.sync_copy(r0, acc.at[pl.ds(s * _NPT + t * 125, 125)])

        plsc.subcore_barrier()

        @pl.loop(0, ph)
        def _(p):
            rb = c * _RPC + s * _LPT + p * cpp
            pltpu.sync_copy(s_hbm.at[pl.ds(rb, cpp)], sidx)
            pltpu.sync_copy(d_hbm.at[pl.ds(rb, cpp)], didx)

            pltpu.async_copy(u_hbm.at[sidx.at[0]], r0, sem0)
            pltpu.async_copy(u_hbm.at[sidx.at[1]], r1, sem1)

            @pl.loop(0, cpp // 2 - 1)
            def _(i):
                a = 2 * i
                pltpu.make_async_copy(u_hbm.at[sidx.at[a]], r0, sem0).wait()
                pltpu.sync_copy(r0, acc.at[didx.at[a]], add=True)
                pltpu.async_copy(u_hbm.at[sidx.at[a + 2]], r0, sem0)
                pltpu.make_async_copy(u_hbm.at[sidx.at[a + 1]], r1, sem1).wait()
                pltpu.sync_copy(r1, acc.at[didx.at[a + 1]], add=True)
                pltpu.async_copy(u_hbm.at[sidx.at[a + 3]], r1, sem1)

            pltpu.make_async_copy(u_hbm.at[sidx.at[cpp - 2]], r0, sem0).wait()
            pltpu.sync_copy(r0, acc.at[didx.at[cpp - 2]], add=True)
            pltpu.make_async_copy(u_hbm.at[sidx.at[cpp - 1]], r1, sem1).wait()
            pltpu.sync_copy(r1, acc.at[didx.at[cpp - 1]], add=True)

        plsc.subcore_barrier()

        @pl.when(s < 15)
        def _():
            pltpu.sync_copy(acc.at[pl.ds(s * 624, 624)],
                            out_hbm.at[c, pl.ds(s * 624, 624)])

        @pl.when(s == 15)
        def _():
            pltpu.sync_copy(acc.at[pl.ds(9360, 640)],
                            out_hbm.at[c, pl.ds(9360, 640)])

    return k(u, src2d, dst2d)


_BLK = 1000


def _row_specs(*minors):
    return [pl.BlockSpec((_BLK, m), lambda i: (i, 0)) for m in minors]


def _mm1(x, w):
    def body(x_ref, w_ref, o_ref):
        o_ref[...] = jnp.dot(x_ref[...], w_ref[...],
                             preferred_element_type=jnp.float32,
                             precision=lax.Precision.HIGHEST)

    return pl.pallas_call(
        body,
        grid=(_N // _BLK,),
        in_specs=_row_specs(_D) + [pl.BlockSpec((_D, _D), lambda i: (0, 0))],
        out_specs=_row_specs(_D)[0],
        out_shape=jax.ShapeDtypeStruct((_N, _D), jnp.float32),
    )(x, w)


def _dinv_col(p0, p1):
    return lax.rsqrt(1.0 + p0[:, 0:1] + p1[:, 0:1])


def _scale(t, p0, p1):
    def body(t_ref, p0_ref, p1_ref, o_ref):
        o_ref[...] = t_ref[...] * _dinv_col(p0_ref[...], p1_ref[...])

    return pl.pallas_call(
        body,
        grid=(_N // _BLK,),
        in_specs=_row_specs(_D, 16, 16),
        out_specs=_row_specs(_D)[0],
        out_shape=jax.ShapeDtypeStruct((_N, _D), jnp.float32),
    )(t, p0, p1)


def _mid(s0, s1, u, p0, p1, b, w):
    def body(s0_ref, s1_ref, u_ref, p0_ref, p1_ref, b_ref, w_ref, o_ref):
        dinv = _dinv_col(p0_ref[...], p1_ref[...])
        z = jnp.maximum(dinv * (s0_ref[...] + s1_ref[...] + u_ref[...])
                        + b_ref[...], 0.0)
        o_ref[...] = dinv * jnp.dot(z, w_ref[...],
                                    preferred_element_type=jnp.float32,
                                    precision=lax.Precision.HIGHEST)

    return pl.pallas_call(
        body,
        grid=(_N // _BLK,),
        in_specs=_row_specs(_D, _D, _D, 16, 16)
        + [pl.BlockSpec((1, _D), lambda i: (0, 0)),
           pl.BlockSpec((_D, _D), lambda i: (0, 0))],
        out_specs=_row_specs(_D)[0],
        out_shape=jax.ShapeDtypeStruct((_N, _D), jnp.float32),
    )(s0, s1, u, p0, p1, b, w)


def _fin(s0, s1, u, p0, p1, b):
    def body(s0_ref, s1_ref, u_ref, p0_ref, p1_ref, b_ref, o_ref):
        dinv = _dinv_col(p0_ref[...], p1_ref[...])
        o_ref[...] = dinv * (s0_ref[...] + s1_ref[...] + u_ref[...]) + b_ref[...]

    return pl.pallas_call(
        body,
        grid=(_N // _BLK,),
        in_specs=_row_specs(_D, _D, _D, 16, 16)
        + [pl.BlockSpec((1, _D), lambda i: (0, 0))],
        out_specs=_row_specs(_D)[0],
        out_shape=jax.ShapeDtypeStruct((_N, _D), jnp.float32),
    )(s0, s1, u, p0, p1, b)


def kernel(x, edge_index, W1, b1, W2, b2, W3, b3):
    src2d = edge_index[0].reshape(_ROWS, _W)
    dst2d = edge_index[1].reshape(_ROWS, _W)
    degp = _deg_sc(dst2d)
    p0, p1 = degp[0], degp[1]
    t1 = _mm1(x, W1)
    u1 = _scale(t1, p0, p1)
    a1 = _agg_sc(u1, src2d, dst2d)
    u2 = _mid(a1[0], a1[1], u1, p0, p1, b1.reshape(1, _D), W2)
    a2 = _agg_sc(u2, src2d, dst2d)
    u3 = _mid(a2[0], a2[1], u2, p0, p1, b2.reshape(1, _D), W3)
    a3 = _agg_sc(u3, src2d, dst2d)
    return _fin(a3[0], a3[1], u3, p0, p1, b3.reshape(1, _D))
